# baseline (speedup 1.0000x reference)
import functools

import jax
import jax.numpy as jnp
from jax import lax
from jax.experimental import pallas as pl
from jax.experimental.pallas import tpu as pltpu
from jax.experimental.pallas import tpu_sc as plsc

_N = 5000
_B = 256
_NP = 5120
_NB = _NP // _B
_MAX_OUT = 1000
_OUT_PAD = 1024

_SC_CORES = 2
_SC_SUBCORES = 16
_SC_WORKERS = _SC_CORES * _SC_SUBCORES
_SC_SLOTS = _OUT_PAD // _SC_WORKERS


def _iou_tile(x1c, y1c, x2c, y2c, ac, x1r, y1r, x2r, y2r, ar):
    xx1 = jnp.maximum(x1c, x1r)
    yy1 = jnp.maximum(y1c, y1r)
    xx2 = jnp.minimum(x2c, x2r)
    yy2 = jnp.minimum(y2c, y2r)
    inter = jnp.clip(xx2 - xx1, 0.0) * jnp.clip(yy2 - yy1, 0.0)
    return inter / (ac + ar - inter + 1e-9)


def _rank_body(srow_ref, scol_ref, srowb_ref, a1_ref, a2_ref):
    p = pl.program_id(0)

    @pl.when(p == 0)
    def _zero():
        a1_ref[:, :] = jnp.zeros((_NB, _B), jnp.float32)

    sc = scol_ref[:, :]
    srb = srowb_ref[:, :]

    lane = lax.broadcasted_iota(jnp.int32, (_B, _B), 1)
    sub = lax.broadcasted_iota(jnp.int32, (_B, _B), 0)
    cd = jnp.where((srb > sc) | ((srb == sc) & (lane < sub)), 1.0, 0.0)
    a2_ref[pl.ds(p * _B, _B), :] = jnp.sum(cd, axis=1, keepdims=True)

    for q in range(1, _NB):
        @pl.when(q > p)
        def _pair():
            sr = srow_ref[0:1, q * _B:(q + 1) * _B]
            c = jnp.where(sr > sc, 1.0, 0.0)
            a1_ref[pl.ds(q, 1), :] = (a1_ref[pl.ds(q, 1), :] + _B
                                      - jnp.sum(c, axis=0, keepdims=True))
            a2_ref[pl.ds(p * _B, _B), :] = (a2_ref[pl.ds(p * _B, _B), :]
                                            + jnp.sum(c, axis=1, keepdims=True))


def _run_rank(srow, scol):
    return pl.pallas_call(
        _rank_body,
        grid=(_NB,),
        in_specs=[
            pl.BlockSpec((1, _NP), lambda p: (0, 0)),
            pl.BlockSpec((_B, 1), lambda p: (p, 0)),
            pl.BlockSpec((1, _B), lambda p: (0, p)),
        ],
        out_specs=[
            pl.BlockSpec((_NB, _B), lambda p: (0, 0)),
            pl.BlockSpec((_NP, 1), lambda p: (0, 0)),
        ],
        out_shape=[
            jax.ShapeDtypeStruct((_NB, _B), jnp.float32),
            jax.ShapeDtypeStruct((_NP, 1), jnp.float32),
        ],
    )(srow, scol, srow)


def _supp_body(thr_ref, cold_ref, rowd_ref, rowb_ref, c_ref, supp_ref,
               cnt_ref):
    p = pl.program_id(0)
    thr = thr_ref[0]

    @pl.when(p == 0)
    def _init():
        gidx = (lax.broadcasted_iota(jnp.int32, (_NB, _B), 0) * _B
                + lax.broadcasted_iota(jnp.int32, (_NB, _B), 1))
        supp_ref[:, :] = jnp.where(gidx < _N, 0.0, 1.0)
        cnt_ref[0] = 0

    cnt0 = cnt_ref[0]

    @pl.when(cnt0 >= _MAX_OUT)
    def _skip():
        c_ref[0, :, :] = jnp.full((1, _B), cnt0, jnp.int32)

    @pl.when(cnt0 < _MAX_OUT)
    def _work():
        x1c = cold_ref[:, 0:1]
        y1c = cold_ref[:, 1:2]
        x2c = cold_ref[:, 2:3]
        y2c = cold_ref[:, 3:4]
        ac = cold_ref[:, 4:5]

        iou_bb = _iou_tile(x1c, y1c, x2c, y2c, ac,
                           rowb_ref[0:1, :], rowb_ref[1:2, :], rowb_ref[2:3, :],
                           rowb_ref[3:4, :], rowb_ref[4:5, :])
        tri = (lax.broadcasted_iota(jnp.int32, (_B, _B), 0)
               < lax.broadcasted_iota(jnp.int32, (_B, _B), 1))
        supbb = jnp.where((iou_bb > thr) & tri, 1.0, 0.0)

        kb0 = 1.0 - supp_ref[pl.ds(p, 1), :]

        def _rcond(carry):
            _, changed = carry
            return changed

        def _rbody(carry):
            kb, _ = carry
            hit = jnp.dot(kb, supbb, preferred_element_type=jnp.float32)
            kb_new = jnp.where(hit > 0.0, 0.0, kb0)
            return kb_new, jnp.any(kb_new != kb)

        kb, _ = lax.while_loop(_rcond, _rbody, (kb0, True))

        tri_le = jnp.where(lax.broadcasted_iota(jnp.int32, (_B, _B), 0)
                           <= lax.broadcasted_iota(jnp.int32, (_B, _B), 1),
                           1.0, 0.0)
        csum = jnp.dot(kb, tri_le, preferred_element_type=jnp.float32)
        c_ref[0, :, :] = csum.astype(jnp.int32) + cnt0
        cnt_ref[0] = cnt0 + jnp.sum(kb).astype(jnp.int32)

        for cb in range(1, _NB):
            @pl.when(cb > p)
            def _prop():
                s = cb * _B
                sf = jnp.where(
                    _iou_tile(x1c, y1c, x2c, y2c, ac,
                              rowd_ref[0:1, s:s + _B], rowd_ref[1:2, s:s + _B],
                              rowd_ref[2:3, s:s + _B], rowd_ref[3:4, s:s + _B],
                              rowd_ref[4:5, s:s + _B]) > thr,
                    1.0, 0.0)
                hits = jnp.dot(kb, sf, preferred_element_type=jnp.float32)
                old = supp_ref[pl.ds(cb, 1), :]
                supp_ref[pl.ds(cb, 1), :] = jnp.maximum(
                    old, jnp.where(hits > 0.0, 1.0, 0.0))


def _run_suppression(coldata, rowdata, thr):
    return pl.pallas_call(
        _supp_body,
        grid=(_NB,),
        in_specs=[
            pl.BlockSpec(memory_space=pltpu.SMEM),
            pl.BlockSpec((_B, 5), lambda p: (p, 0)),
            pl.BlockSpec((5, _NP), lambda p: (0, 0)),
            pl.BlockSpec((5, _B), lambda p: (0, p)),
        ],
        out_specs=pl.BlockSpec((1, 1, _B), lambda p: (p, 0, 0)),
        out_shape=jax.ShapeDtypeStruct((_NB, 1, _B), jnp.int32),
        scratch_shapes=[
            pltpu.VMEM((_NB, _B), jnp.float32),
            pltpu.SMEM((1,), jnp.int32),
        ],
    )(thr, coldata, rowdata, rowdata)


def _compact_body(c_hbm, orig_hbm, out_hbm, c_v, o_v, res_v):
    wid = lax.axis_index("s") * _SC_CORES + lax.axis_index("c")
    pltpu.sync_copy(c_hbm, c_v)
    pltpu.sync_copy(orig_hbm, o_v)
    last_idx = jnp.full((16,), _NP - 1, jnp.int32)
    c_last = plsc.load_gather(c_v, [last_idx])
    for g in range(_SC_SLOTS // 16):
        tgt = wid * _SC_SLOTS + g * 16 + lax.iota(jnp.int32, 16) + 1

        def _bstep(_, carry):
            lo, hi = carry
            live = lo < hi
            mid = jnp.minimum((lo + hi) // 2, _NP - 1)
            v = plsc.load_gather(c_v, [mid])
            pred = v >= tgt
            lo2 = jnp.where(live & jnp.logical_not(pred), mid + 1, lo)
            hi2 = jnp.where(live & pred, mid, hi)
            return lo2, hi2

        lo0 = jnp.zeros((16,), jnp.int32)
        hi0 = jnp.full((16,), _NP, jnp.int32)
        pos, _ = lax.fori_loop(0, 13, _bstep, (lo0, hi0))
        valid = c_last >= tgt
        safe = jnp.minimum(pos, _NP - 1)
        ov = plsc.load_gather(o_v, [safe])
        res_v[pl.ds(g * 16, 16)] = jnp.where(valid, ov, -1)
    pltpu.sync_copy(res_v, out_hbm.at[pl.ds(wid * _SC_SLOTS, _SC_SLOTS)])


_CH = _NP // _SC_WORKERS


def _scatter_body(r1_hbm, r2_hbm, x1_hbm, y1_hbm, x2_hbm, y2_hbm,
                  sx1_hbm, sy1_hbm, sx2_hbm, sy2_hbm, sar_hbm, sorig_hbm,
                  x1v, y1v, x2v, y2v, r1v, r2v, av, ov, riv):
    wid = lax.axis_index("s") * _SC_CORES + lax.axis_index("c")
    base = wid * _CH
    pltpu.sync_copy(r1_hbm.at[pl.ds(base, _CH)], r1v)
    pltpu.sync_copy(r2_hbm.at[pl.ds(base, _CH)], r2v)
    pltpu.sync_copy(x1_hbm.at[pl.ds(base, _CH)], x1v)
    pltpu.sync_copy(y1_hbm.at[pl.ds(base, _CH)], y1v)
    pltpu.sync_copy(x2_hbm.at[pl.ds(base, _CH)], x2v)
    pltpu.sync_copy(y2_hbm.at[pl.ds(base, _CH)], y2v)
    for k in range(_CH // 16):
        sl = pl.ds(k * 16, 16)
        riv[k // 5, pl.ds((k % 5) * 16, 16)] = (
            r1v[sl] + r2v[sl]).astype(jnp.int32)
        av[sl] = (x2v[sl] - x1v[sl]) * (y2v[sl] - y1v[sl])
        ov[sl] = base + k * 16 + lax.iota(jnp.int32, 16)
    for g in range(2):
        idx = riv.at[g]
        s80 = pl.ds(g * 80, 80)
        pltpu.sync_copy(x1v.at[s80], sx1_hbm.at[idx])
        pltpu.sync_copy(y1v.at[s80], sy1_hbm.at[idx])
        pltpu.sync_copy(x2v.at[s80], sx2_hbm.at[idx])
        pltpu.sync_copy(y2v.at[s80], sy2_hbm.at[idx])
        pltpu.sync_copy(av.at[s80], sar_hbm.at[idx])
        pltpu.sync_copy(ov.at[s80], sorig_hbm.at[idx])


@functools.cache
def _scatter_call():
    f32 = jnp.float32
    return pl.kernel(
        _scatter_body,
        out_type=[jax.ShapeDtypeStruct((_NP,), f32)] * 5
        + [jax.ShapeDtypeStruct((_NP,), jnp.int32)],
        mesh=plsc.VectorSubcoreMesh(core_axis_name="c", subcore_axis_name="s"),
        compiler_params=pltpu.CompilerParams(needs_layout_passes=False),
        scratch_types=[
            pltpu.VMEM((_CH,), f32),
            pltpu.VMEM((_CH,), f32),
            pltpu.VMEM((_CH,), f32),
            pltpu.VMEM((_CH,), f32),
            pltpu.VMEM((_CH,), f32),
            pltpu.VMEM((_CH,), f32),
            pltpu.VMEM((_CH,), f32),
            pltpu.VMEM((_CH,), jnp.int32),
            pltpu.VMEM((2, 80), jnp.int32),
        ],
    )


@functools.cache
def _compact_call():
    return pl.kernel(
        _compact_body,
        out_type=jax.ShapeDtypeStruct((_OUT_PAD,), jnp.int32),
        mesh=plsc.VectorSubcoreMesh(core_axis_name="c", subcore_axis_name="s"),
        compiler_params=pltpu.CompilerParams(needs_layout_passes=False),
        scratch_types=[
            pltpu.VMEM((_NP,), jnp.int32),
            pltpu.VMEM((_NP,), jnp.int32),
            pltpu.VMEM((_SC_SLOTS,), jnp.int32),
        ],
    )


def kernel(boxes, scores, iou_threshold):
    pad = _NP - _N
    sp = jnp.pad(scores.astype(jnp.float32), (0, pad), constant_values=-1.0)
    a1, a2 = _run_rank(sp.reshape(1, _NP), sp.reshape(_NP, 1))
    x1u = jnp.pad(boxes[:, 0], (0, pad))
    y1u = jnp.pad(boxes[:, 1], (0, pad))
    x2u = jnp.pad(boxes[:, 2], (0, pad))
    y2u = jnp.pad(boxes[:, 3], (0, pad))
    sx1, sy1, sx2, sy2, sar, sorig = _scatter_call()(
        a1.reshape(_NP), a2.reshape(_NP), x1u, y1u, x2u, y2u)
    rowdata = jnp.stack([sx1, sy1, sx2, sy2, sar])
    coldata = rowdata.T
    thr = jnp.reshape(iou_threshold.astype(jnp.float32), (1,))

    c = _run_suppression(coldata, rowdata, thr).reshape(_NP)
    out = _compact_call()(c, sorig)
    return out[:_MAX_OUT]


# speedup vs baseline: 1.0046x; 1.0046x over previous
import functools

import jax
import jax.numpy as jnp
from jax import lax
from jax.experimental import pallas as pl
from jax.experimental.pallas import tpu as pltpu
from jax.experimental.pallas import tpu_sc as plsc

_N = 5000
_B = 256
_NP = 5120
_NB = _NP // _B
_MAX_OUT = 1000
_OUT_PAD = 1024

_SC_CORES = 2
_SC_SUBCORES = 16
_SC_WORKERS = _SC_CORES * _SC_SUBCORES
_SC_SLOTS = _OUT_PAD // _SC_WORKERS


def _iou_tile(x1c, y1c, x2c, y2c, ac, x1r, y1r, x2r, y2r, ar):
    xx1 = jnp.maximum(x1c, x1r)
    yy1 = jnp.maximum(y1c, y1r)
    xx2 = jnp.minimum(x2c, x2r)
    yy2 = jnp.minimum(y2c, y2r)
    inter = jnp.clip(xx2 - xx1, 0.0) * jnp.clip(yy2 - yy1, 0.0)
    return inter / (ac + ar - inter + 1e-9)


def _rank_body(srow_ref, scol_ref, srowb_ref, a1_ref, a2_ref):
    p = pl.program_id(0)

    @pl.when(p == 0)
    def _zero():
        a1_ref[:, :] = jnp.zeros((_NB, _B), jnp.float32)

    sc = scol_ref[:, :]
    srb = srowb_ref[:, :]

    lane = lax.broadcasted_iota(jnp.int32, (_B, _B), 1)
    sub = lax.broadcasted_iota(jnp.int32, (_B, _B), 0)
    cd = jnp.where((srb > sc) | ((srb == sc) & (lane < sub)), 1.0, 0.0)
    a2_ref[pl.ds(p * _B, _B), :] = jnp.sum(cd, axis=1, keepdims=True)

    for q in range(1, _NB):
        @pl.when(q > p)
        def _pair():
            sr = srow_ref[0:1, q * _B:(q + 1) * _B]
            c = jnp.where(sr > sc, 1.0, 0.0)
            a1_ref[pl.ds(q, 1), :] = (a1_ref[pl.ds(q, 1), :] + _B
                                      - jnp.sum(c, axis=0, keepdims=True))
            a2_ref[pl.ds(p * _B, _B), :] = (a2_ref[pl.ds(p * _B, _B), :]
                                            + jnp.sum(c, axis=1, keepdims=True))


def _run_rank(srow, scol):
    return pl.pallas_call(
        _rank_body,
        grid=(_NB,),
        in_specs=[
            pl.BlockSpec((1, _NP), lambda p: (0, 0)),
            pl.BlockSpec((_B, 1), lambda p: (p, 0)),
            pl.BlockSpec((1, _B), lambda p: (0, p)),
        ],
        out_specs=[
            pl.BlockSpec((_NB, _B), lambda p: (0, 0)),
            pl.BlockSpec((_NP, 1), lambda p: (0, 0)),
        ],
        out_shape=[
            jax.ShapeDtypeStruct((_NB, _B), jnp.float32),
            jax.ShapeDtypeStruct((_NP, 1), jnp.float32),
        ],
    )(srow, scol, srow)


def _supp_body(thr_ref, cold_ref, rowd_ref, rowb_ref, c_ref, supp_ref,
               cnt_ref):
    p = pl.program_id(0)
    thr = thr_ref[0]

    @pl.when(p == 0)
    def _init():
        gidx = (lax.broadcasted_iota(jnp.int32, (_NB, _B), 0) * _B
                + lax.broadcasted_iota(jnp.int32, (_NB, _B), 1))
        supp_ref[:, :] = jnp.where(gidx < _N, 0.0, 1.0)
        cnt_ref[0] = 0

    cnt0 = cnt_ref[0]

    @pl.when(cnt0 >= _MAX_OUT)
    def _skip():
        c_ref[0, :, :] = jnp.full((1, _B), cnt0, jnp.int32)

    @pl.when(cnt0 < _MAX_OUT)
    def _work():
        x1c = cold_ref[:, 0:1]
        y1c = cold_ref[:, 1:2]
        x2c = cold_ref[:, 2:3]
        y2c = cold_ref[:, 3:4]
        ac = cold_ref[:, 4:5]

        iou_bb = _iou_tile(x1c, y1c, x2c, y2c, ac,
                           rowb_ref[0:1, :], rowb_ref[1:2, :], rowb_ref[2:3, :],
                           rowb_ref[3:4, :], rowb_ref[4:5, :])
        tri = (lax.broadcasted_iota(jnp.int32, (_B, _B), 0)
               < lax.broadcasted_iota(jnp.int32, (_B, _B), 1))
        supbb = jnp.where((iou_bb > thr) & tri, 1.0, 0.0)

        kb0 = 1.0 - supp_ref[pl.ds(p, 1), :]

        def _rcond(carry):
            _, changed = carry
            return changed

        def _rbody(carry):
            kb, _ = carry
            hit = jnp.dot(kb, supbb, preferred_element_type=jnp.float32)
            kb_new = jnp.where(hit > 0.0, 0.0, kb0)
            return kb_new, jnp.any(kb_new != kb)

        kb, _ = lax.while_loop(_rcond, _rbody, (kb0, True))

        tri_le = jnp.where(lax.broadcasted_iota(jnp.int32, (_B, _B), 0)
                           <= lax.broadcasted_iota(jnp.int32, (_B, _B), 1),
                           1.0, 0.0)
        csum = jnp.dot(kb, tri_le, preferred_element_type=jnp.float32)
        c_ref[0, :, :] = csum.astype(jnp.int32) + cnt0
        cnt_ref[0] = cnt0 + jnp.sum(kb).astype(jnp.int32)

        for cb in range(1, _NB):
            @pl.when(cb > p)
            def _prop():
                s = cb * _B
                sf = jnp.where(
                    _iou_tile(x1c, y1c, x2c, y2c, ac,
                              rowd_ref[0:1, s:s + _B], rowd_ref[1:2, s:s + _B],
                              rowd_ref[2:3, s:s + _B], rowd_ref[3:4, s:s + _B],
                              rowd_ref[4:5, s:s + _B]) > thr,
                    1.0, 0.0)
                hits = jnp.dot(kb, sf, preferred_element_type=jnp.float32)
                old = supp_ref[pl.ds(cb, 1), :]
                supp_ref[pl.ds(cb, 1), :] = jnp.maximum(
                    old, jnp.where(hits > 0.0, 1.0, 0.0))


def _run_suppression(coldata, rowdata, thr):
    return pl.pallas_call(
        _supp_body,
        grid=(_NB,),
        in_specs=[
            pl.BlockSpec(memory_space=pltpu.SMEM),
            pl.BlockSpec((_B, 5), lambda p: (p, 0)),
            pl.BlockSpec((5, _NP), lambda p: (0, 0)),
            pl.BlockSpec((5, _B), lambda p: (0, p)),
        ],
        out_specs=pl.BlockSpec((1, 1, _B), lambda p: (p, 0, 0)),
        out_shape=jax.ShapeDtypeStruct((_NB, 1, _B), jnp.int32),
        scratch_shapes=[
            pltpu.VMEM((_NB, _B), jnp.float32),
            pltpu.SMEM((1,), jnp.int32),
        ],
    )(thr, coldata, rowdata, rowdata)


def _compact_body(c_hbm, orig_hbm, out_hbm, c_v, o_v, res_v):
    wid = lax.axis_index("s") * _SC_CORES + lax.axis_index("c")
    pltpu.sync_copy(c_hbm, c_v)
    pltpu.sync_copy(orig_hbm, o_v)
    last_idx = jnp.full((16,), _NP - 1, jnp.int32)
    c_last = plsc.load_gather(c_v, [last_idx])
    for g in range(_SC_SLOTS // 16):
        tgt = wid * _SC_SLOTS + g * 16 + lax.iota(jnp.int32, 16) + 1

        def _bstep(_, carry):
            lo, hi = carry
            live = lo < hi
            mid = jnp.minimum((lo + hi) // 2, _NP - 1)
            v = plsc.load_gather(c_v, [mid])
            pred = v >= tgt
            lo2 = jnp.where(live & jnp.logical_not(pred), mid + 1, lo)
            hi2 = jnp.where(live & pred, mid, hi)
            return lo2, hi2

        lo0 = jnp.zeros((16,), jnp.int32)
        hi0 = jnp.full((16,), _NP, jnp.int32)
        pos, _ = lax.fori_loop(0, 13, _bstep, (lo0, hi0))
        valid = c_last >= tgt
        safe = jnp.minimum(pos, _NP - 1)
        ov = plsc.load_gather(o_v, [safe])
        res_v[pl.ds(g * 16, 16)] = jnp.where(valid, ov, -1)
    pltpu.sync_copy(res_v, out_hbm.at[pl.ds(wid * _SC_SLOTS, _SC_SLOTS)])


_CH = _NP // _SC_WORKERS


def _scatter_body(r1_hbm, r2_hbm, x1_hbm, y1_hbm, x2_hbm, y2_hbm,
                  sx1_hbm, sy1_hbm, sx2_hbm, sy2_hbm, sar_hbm, sorig_hbm,
                  x1v, y1v, x2v, y2v, r1v, r2v, av, ov, riv, sem):
    wid = lax.axis_index("s") * _SC_CORES + lax.axis_index("c")
    base = wid * _CH
    src = pl.ds(base, _CH)
    loads = [pltpu.async_copy(r1_hbm.at[src], r1v, sem),
             pltpu.async_copy(r2_hbm.at[src], r2v, sem),
             pltpu.async_copy(x1_hbm.at[src], x1v, sem),
             pltpu.async_copy(y1_hbm.at[src], y1v, sem),
             pltpu.async_copy(x2_hbm.at[src], x2v, sem),
             pltpu.async_copy(y2_hbm.at[src], y2v, sem)]
    for h in loads:
        h.wait()
    for k in range(_CH // 16):
        sl = pl.ds(k * 16, 16)
        riv[k // 5, pl.ds((k % 5) * 16, 16)] = (
            r1v[sl] + r2v[sl]).astype(jnp.int32)
        av[sl] = (x2v[sl] - x1v[sl]) * (y2v[sl] - y1v[sl])
        ov[sl] = base + k * 16 + lax.iota(jnp.int32, 16)
    stores = []
    for g in range(2):
        idx = riv.at[g]
        s80 = pl.ds(g * 80, 80)
        stores.append(pltpu.async_copy(x1v.at[s80], sx1_hbm.at[idx], sem))
        stores.append(pltpu.async_copy(y1v.at[s80], sy1_hbm.at[idx], sem))
        stores.append(pltpu.async_copy(x2v.at[s80], sx2_hbm.at[idx], sem))
        stores.append(pltpu.async_copy(y2v.at[s80], sy2_hbm.at[idx], sem))
        stores.append(pltpu.async_copy(av.at[s80], sar_hbm.at[idx], sem))
        stores.append(pltpu.async_copy(ov.at[s80], sorig_hbm.at[idx], sem))
    for h in stores:
        h.wait()


@functools.cache
def _scatter_call():
    f32 = jnp.float32
    return pl.kernel(
        _scatter_body,
        out_type=[jax.ShapeDtypeStruct((_NP,), f32)] * 5
        + [jax.ShapeDtypeStruct((_NP,), jnp.int32)],
        mesh=plsc.VectorSubcoreMesh(core_axis_name="c", subcore_axis_name="s"),
        compiler_params=pltpu.CompilerParams(needs_layout_passes=False),
        scratch_types=[
            pltpu.VMEM((_CH,), f32),
            pltpu.VMEM((_CH,), f32),
            pltpu.VMEM((_CH,), f32),
            pltpu.VMEM((_CH,), f32),
            pltpu.VMEM((_CH,), f32),
            pltpu.VMEM((_CH,), f32),
            pltpu.VMEM((_CH,), f32),
            pltpu.VMEM((_CH,), jnp.int32),
            pltpu.VMEM((2, 80), jnp.int32),
            pltpu.SemaphoreType.DMA,
        ],
    )


@functools.cache
def _compact_call():
    return pl.kernel(
        _compact_body,
        out_type=jax.ShapeDtypeStruct((_OUT_PAD,), jnp.int32),
        mesh=plsc.VectorSubcoreMesh(core_axis_name="c", subcore_axis_name="s"),
        compiler_params=pltpu.CompilerParams(needs_layout_passes=False),
        scratch_types=[
            pltpu.VMEM((_NP,), jnp.int32),
            pltpu.VMEM((_NP,), jnp.int32),
            pltpu.VMEM((_SC_SLOTS,), jnp.int32),
        ],
    )


def kernel(boxes, scores, iou_threshold):
    pad = _NP - _N
    sp = jnp.pad(scores.astype(jnp.float32), (0, pad), constant_values=-1.0)
    a1, a2 = _run_rank(sp.reshape(1, _NP), sp.reshape(_NP, 1))
    x1u = jnp.pad(boxes[:, 0], (0, pad))
    y1u = jnp.pad(boxes[:, 1], (0, pad))
    x2u = jnp.pad(boxes[:, 2], (0, pad))
    y2u = jnp.pad(boxes[:, 3], (0, pad))
    sx1, sy1, sx2, sy2, sar, sorig = _scatter_call()(
        a1.reshape(_NP), a2.reshape(_NP), x1u, y1u, x2u, y2u)
    rowdata = jnp.stack([sx1, sy1, sx2, sy2, sar])
    coldata = rowdata.T
    thr = jnp.reshape(iou_threshold.astype(jnp.float32), (1,))

    c = _run_suppression(coldata, rowdata, thr).reshape(_NP)
    out = _compact_call()(c, sorig)
    return out[:_MAX_OUT]


# speedup vs baseline: 1.1023x; 1.0972x over previous
import functools

import jax
import jax.numpy as jnp
from jax import lax
from jax.experimental import pallas as pl
from jax.experimental.pallas import tpu as pltpu
from jax.experimental.pallas import tpu_sc as plsc

_N = 5000
_B = 256
_NP = 5120
_NB = _NP // _B
_MAX_OUT = 1000
_OUT_PAD = 1024

_SC_CORES = 2
_SC_SUBCORES = 16
_SC_WORKERS = _SC_CORES * _SC_SUBCORES
_SC_SLOTS = _OUT_PAD // _SC_WORKERS


def _iou_tile(x1c, y1c, x2c, y2c, ac, x1r, y1r, x2r, y2r, ar):
    xx1 = jnp.maximum(x1c, x1r)
    yy1 = jnp.maximum(y1c, y1r)
    xx2 = jnp.minimum(x2c, x2r)
    yy2 = jnp.minimum(y2c, y2r)
    inter = jnp.clip(xx2 - xx1, 0.0) * jnp.clip(yy2 - yy1, 0.0)
    return inter / (ac + ar - inter + 1e-9)


def _rank_body(srow_ref, scol_ref, srowb_ref, a1_ref, a2_ref):
    p = pl.program_id(0)

    @pl.when(p == 0)
    def _zero():
        a1_ref[:, :] = jnp.zeros((_NB, _B), jnp.float32)

    sc = scol_ref[:, :]
    srb = srowb_ref[:, :]

    lane = lax.broadcasted_iota(jnp.int32, (_B, _B), 1)
    sub = lax.broadcasted_iota(jnp.int32, (_B, _B), 0)
    cd = jnp.where((srb > sc) | ((srb == sc) & (lane < sub)), 1.0, 0.0)
    a2_ref[pl.ds(p * _B, _B), :] = jnp.sum(cd, axis=1, keepdims=True)

    for q in range(1, _NB):
        @pl.when(q > p)
        def _pair():
            sr = srow_ref[0:1, q * _B:(q + 1) * _B]
            c = jnp.where(sr > sc, 1.0, 0.0)
            a1_ref[pl.ds(q, 1), :] = (a1_ref[pl.ds(q, 1), :] + _B
                                      - jnp.sum(c, axis=0, keepdims=True))
            a2_ref[pl.ds(p * _B, _B), :] = (a2_ref[pl.ds(p * _B, _B), :]
                                            + jnp.sum(c, axis=1, keepdims=True))


def _run_rank(srow, scol):
    return pl.pallas_call(
        _rank_body,
        grid=(_NB,),
        in_specs=[
            pl.BlockSpec((1, _NP), lambda p: (0, 0)),
            pl.BlockSpec((_B, 1), lambda p: (p, 0)),
            pl.BlockSpec((1, _B), lambda p: (0, p)),
        ],
        out_specs=[
            pl.BlockSpec((_NB, _B), lambda p: (0, 0)),
            pl.BlockSpec((_NP, 1), lambda p: (0, 0)),
        ],
        out_shape=[
            jax.ShapeDtypeStruct((_NB, _B), jnp.float32),
            jax.ShapeDtypeStruct((_NP, 1), jnp.float32),
        ],
    )(srow, scol, srow)


def _supp_body(thr_ref, coldf_ref, cold_ref, rowb_ref, c_ref, keep_ref,
               ext_ref, cnt_ref):
    p = pl.program_id(0)
    thr = thr_ref[0]

    @pl.when(p == 0)
    def _init():
        cnt_ref[0] = 0

    cnt0 = cnt_ref[0]

    @pl.when(cnt0 >= _MAX_OUT)
    def _skip():
        c_ref[0, :, :] = jnp.full((1, _B), cnt0, jnp.int32)

    @pl.when(cnt0 < _MAX_OUT)
    def _work():
        x1r = rowb_ref[0:1, :]
        y1r = rowb_ref[1:2, :]
        x2r = rowb_ref[2:3, :]
        y2r = rowb_ref[3:4, :]
        ar = rowb_ref[4:5, :]

        ext_ref[:, :] = jnp.zeros((1, _B), jnp.float32)
        for q in range(_NB - 1):
            @pl.when(q < p)
            def _pull(q=q):
                s = q * _B
                sf = jnp.where(
                    _iou_tile(coldf_ref[pl.ds(s, _B), 0:1],
                              coldf_ref[pl.ds(s, _B), 1:2],
                              coldf_ref[pl.ds(s, _B), 2:3],
                              coldf_ref[pl.ds(s, _B), 3:4],
                              coldf_ref[pl.ds(s, _B), 4:5],
                              x1r, y1r, x2r, y2r, ar) > thr,
                    1.0, 0.0)
                kq = keep_ref[pl.ds(q, 1), :]
                hits = jnp.dot(kq, sf, preferred_element_type=jnp.float32)
                ext_ref[:, :] = jnp.maximum(ext_ref[:, :], hits)

        lane = lax.broadcasted_iota(jnp.int32, (1, _B), 1)
        valid = jnp.where(p * _B + lane < _N, 1.0, 0.0)
        kb0 = jnp.where(ext_ref[:, :] > 0.0, 0.0, valid)

        x1c = cold_ref[:, 0:1]
        y1c = cold_ref[:, 1:2]
        x2c = cold_ref[:, 2:3]
        y2c = cold_ref[:, 3:4]
        ac = cold_ref[:, 4:5]
        iou_bb = _iou_tile(x1c, y1c, x2c, y2c, ac, x1r, y1r, x2r, y2r, ar)
        tri = (lax.broadcasted_iota(jnp.int32, (_B, _B), 0)
               < lax.broadcasted_iota(jnp.int32, (_B, _B), 1))
        supbb = jnp.where((iou_bb > thr) & tri, 1.0, 0.0)

        def _rcond(carry):
            _, changed = carry
            return changed

        def _rbody(carry):
            kb, _ = carry
            hit = jnp.dot(kb, supbb, preferred_element_type=jnp.float32)
            kb_new = jnp.where(hit > 0.0, 0.0, kb0)
            return kb_new, jnp.any(kb_new != kb)

        kb, _ = lax.while_loop(_rcond, _rbody, (kb0, True))
        keep_ref[pl.ds(p, 1), :] = kb

        tri_le = jnp.where(lax.broadcasted_iota(jnp.int32, (_B, _B), 0)
                           <= lax.broadcasted_iota(jnp.int32, (_B, _B), 1),
                           1.0, 0.0)
        csum = jnp.dot(kb, tri_le, preferred_element_type=jnp.float32)
        c_ref[0, :, :] = csum.astype(jnp.int32) + cnt0
        cnt_ref[0] = cnt0 + jnp.sum(kb).astype(jnp.int32)


def _run_suppression(coldata, rowdata, thr):
    return pl.pallas_call(
        _supp_body,
        grid=(_NB,),
        in_specs=[
            pl.BlockSpec(memory_space=pltpu.SMEM),
            pl.BlockSpec((_NP, 5), lambda p: (0, 0)),
            pl.BlockSpec((_B, 5), lambda p: (p, 0)),
            pl.BlockSpec((5, _B), lambda p: (0, p)),
        ],
        out_specs=pl.BlockSpec((1, 1, _B), lambda p: (p, 0, 0)),
        out_shape=jax.ShapeDtypeStruct((_NB, 1, _B), jnp.int32),
        scratch_shapes=[
            pltpu.VMEM((_NB, _B), jnp.float32),
            pltpu.VMEM((1, _B), jnp.float32),
            pltpu.SMEM((1,), jnp.int32),
        ],
    )(thr, coldata, coldata, rowdata)


def _compact_body(c_hbm, orig_hbm, out_hbm, c_v, o_v, res_v):
    wid = lax.axis_index("s") * _SC_CORES + lax.axis_index("c")
    pltpu.sync_copy(c_hbm, c_v)
    pltpu.sync_copy(orig_hbm, o_v)
    last_idx = jnp.full((16,), _NP - 1, jnp.int32)
    c_last = plsc.load_gather(c_v, [last_idx])
    for g in range(_SC_SLOTS // 16):
        tgt = wid * _SC_SLOTS + g * 16 + lax.iota(jnp.int32, 16) + 1

        def _bstep(_, carry):
            lo, hi = carry
            live = lo < hi
            mid = jnp.minimum((lo + hi) // 2, _NP - 1)
            v = plsc.load_gather(c_v, [mid])
            pred = v >= tgt
            lo2 = jnp.where(live & jnp.logical_not(pred), mid + 1, lo)
            hi2 = jnp.where(live & pred, mid, hi)
            return lo2, hi2

        lo0 = jnp.zeros((16,), jnp.int32)
        hi0 = jnp.full((16,), _NP, jnp.int32)
        pos, _ = lax.fori_loop(0, 13, _bstep, (lo0, hi0))
        valid = c_last >= tgt
        safe = jnp.minimum(pos, _NP - 1)
        ov = plsc.load_gather(o_v, [safe])
        res_v[pl.ds(g * 16, 16)] = jnp.where(valid, ov, -1)
    pltpu.sync_copy(res_v, out_hbm.at[pl.ds(wid * _SC_SLOTS, _SC_SLOTS)])


_CH = _NP // _SC_WORKERS


def _scatter_body(r1_hbm, r2_hbm, x1_hbm, y1_hbm, x2_hbm, y2_hbm,
                  sx1_hbm, sy1_hbm, sx2_hbm, sy2_hbm, sar_hbm, sorig_hbm,
                  x1v, y1v, x2v, y2v, r1v, r2v, av, ov, riv, sem):
    wid = lax.axis_index("s") * _SC_CORES + lax.axis_index("c")
    base = wid * _CH
    src = pl.ds(base, _CH)
    loads = [pltpu.async_copy(r1_hbm.at[src], r1v, sem),
             pltpu.async_copy(r2_hbm.at[src], r2v, sem),
             pltpu.async_copy(x1_hbm.at[src], x1v, sem),
             pltpu.async_copy(y1_hbm.at[src], y1v, sem),
             pltpu.async_copy(x2_hbm.at[src], x2v, sem),
             pltpu.async_copy(y2_hbm.at[src], y2v, sem)]
    for h in loads:
        h.wait()
    for k in range(_CH // 16):
        sl = pl.ds(k * 16, 16)
        riv[k // 5, pl.ds((k % 5) * 16, 16)] = (
            r1v[sl] + r2v[sl]).astype(jnp.int32)
        av[sl] = (x2v[sl] - x1v[sl]) * (y2v[sl] - y1v[sl])
        ov[sl] = base + k * 16 + lax.iota(jnp.int32, 16)
    stores = []
    for g in range(2):
        idx = riv.at[g]
        s80 = pl.ds(g * 80, 80)
        stores.append(pltpu.async_copy(x1v.at[s80], sx1_hbm.at[idx], sem))
        stores.append(pltpu.async_copy(y1v.at[s80], sy1_hbm.at[idx], sem))
        stores.append(pltpu.async_copy(x2v.at[s80], sx2_hbm.at[idx], sem))
        stores.append(pltpu.async_copy(y2v.at[s80], sy2_hbm.at[idx], sem))
        stores.append(pltpu.async_copy(av.at[s80], sar_hbm.at[idx], sem))
        stores.append(pltpu.async_copy(ov.at[s80], sorig_hbm.at[idx], sem))
    for h in stores:
        h.wait()


@functools.cache
def _scatter_call():
    f32 = jnp.float32
    return pl.kernel(
        _scatter_body,
        out_type=[jax.ShapeDtypeStruct((_NP,), f32)] * 5
        + [jax.ShapeDtypeStruct((_NP,), jnp.int32)],
        mesh=plsc.VectorSubcoreMesh(core_axis_name="c", subcore_axis_name="s"),
        compiler_params=pltpu.CompilerParams(needs_layout_passes=False),
        scratch_types=[
            pltpu.VMEM((_CH,), f32),
            pltpu.VMEM((_CH,), f32),
            pltpu.VMEM((_CH,), f32),
            pltpu.VMEM((_CH,), f32),
            pltpu.VMEM((_CH,), f32),
            pltpu.VMEM((_CH,), f32),
            pltpu.VMEM((_CH,), f32),
            pltpu.VMEM((_CH,), jnp.int32),
            pltpu.VMEM((2, 80), jnp.int32),
            pltpu.SemaphoreType.DMA,
        ],
    )


@functools.cache
def _compact_call():
    return pl.kernel(
        _compact_body,
        out_type=jax.ShapeDtypeStruct((_OUT_PAD,), jnp.int32),
        mesh=plsc.VectorSubcoreMesh(core_axis_name="c", subcore_axis_name="s"),
        compiler_params=pltpu.CompilerParams(needs_layout_passes=False),
        scratch_types=[
            pltpu.VMEM((_NP,), jnp.int32),
            pltpu.VMEM((_NP,), jnp.int32),
            pltpu.VMEM((_SC_SLOTS,), jnp.int32),
        ],
    )


def kernel(boxes, scores, iou_threshold):
    pad = _NP - _N
    sp = jnp.pad(scores.astype(jnp.float32), (0, pad), constant_values=-1.0)
    a1, a2 = _run_rank(sp.reshape(1, _NP), sp.reshape(_NP, 1))
    x1u = jnp.pad(boxes[:, 0], (0, pad))
    y1u = jnp.pad(boxes[:, 1], (0, pad))
    x2u = jnp.pad(boxes[:, 2], (0, pad))
    y2u = jnp.pad(boxes[:, 3], (0, pad))
    sx1, sy1, sx2, sy2, sar, sorig = _scatter_call()(
        a1.reshape(_NP), a2.reshape(_NP), x1u, y1u, x2u, y2u)
    rowdata = jnp.stack([sx1, sy1, sx2, sy2, sar])
    coldata = rowdata.T
    thr = jnp.reshape(iou_threshold.astype(jnp.float32), (1,))

    c = _run_suppression(coldata, rowdata, thr).reshape(_NP)
    out = _compact_call()(c, sorig)
    return out[:_MAX_OUT]


# speedup vs baseline: 2.1584x; 1.9582x over previous
import functools

import jax
import jax.numpy as jnp
from jax import lax
from jax.experimental import pallas as pl
from jax.experimental.pallas import tpu as pltpu
from jax.experimental.pallas import tpu_sc as plsc

_N = 5000
_B = 256
_NP = 5120
_NB = _NP // _B
_MAX_OUT = 1000
_OUT_PAD = 1024

_SC_CORES = 2
_SC_SUBCORES = 16
_SC_WORKERS = _SC_CORES * _SC_SUBCORES
_SC_SLOTS = _OUT_PAD // _SC_WORKERS


def _iou_tile(x1c, y1c, x2c, y2c, ac, x1r, y1r, x2r, y2r, ar):
    xx1 = jnp.maximum(x1c, x1r)
    yy1 = jnp.maximum(y1c, y1r)
    xx2 = jnp.minimum(x2c, x2r)
    yy2 = jnp.minimum(y2c, y2r)
    inter = jnp.clip(xx2 - xx1, 0.0) * jnp.clip(yy2 - yy1, 0.0)
    return inter / (ac + ar - inter + 1e-9)


def _rank_body(srow_ref, scol_ref, srowb_ref, a1_ref, a2_ref):
    p = pl.program_id(0)

    @pl.when(p == 0)
    def _zero():
        a1_ref[:, :] = jnp.zeros((_NB, _B), jnp.float32)

    sc = scol_ref[:, :]
    srb = srowb_ref[:, :]

    lane = lax.broadcasted_iota(jnp.int32, (_B, _B), 1)
    sub = lax.broadcasted_iota(jnp.int32, (_B, _B), 0)
    cd = jnp.where((srb > sc) | ((srb == sc) & (lane < sub)), 1.0, 0.0)
    a2_ref[pl.ds(p * _B, _B), :] = jnp.sum(cd, axis=1, keepdims=True)

    for q in range(1, _NB):
        @pl.when(q > p)
        def _pair():
            sr = srow_ref[0:1, q * _B:(q + 1) * _B]
            c = jnp.where(sr > sc, 1.0, 0.0)
            a1_ref[pl.ds(q, 1), :] = (a1_ref[pl.ds(q, 1), :] + _B
                                      - jnp.sum(c, axis=0, keepdims=True))
            a2_ref[pl.ds(p * _B, _B), :] = (a2_ref[pl.ds(p * _B, _B), :]
                                            + jnp.sum(c, axis=1, keepdims=True))


def _run_rank(srow, scol):
    return pl.pallas_call(
        _rank_body,
        grid=(_NB,),
        in_specs=[
            pl.BlockSpec((1, _NP), lambda p: (0, 0)),
            pl.BlockSpec((_B, 1), lambda p: (p, 0)),
            pl.BlockSpec((1, _B), lambda p: (0, p)),
        ],
        out_specs=[
            pl.BlockSpec((_NB, _B), lambda p: (0, 0)),
            pl.BlockSpec((_NP, 1), lambda p: (0, 0)),
        ],
        out_shape=[
            jax.ShapeDtypeStruct((_NB, _B), jnp.float32),
            jax.ShapeDtypeStruct((_NP, 1), jnp.float32),
        ],
    )(srow, scol, srow)


def _supp_body(thr_ref, coldf_ref, cold_ref, rowb_ref, c_ref, keep_ref,
               ext_ref, cnt_ref):
    p = pl.program_id(0)
    thr = thr_ref[0]

    @pl.when(p == 0)
    def _init():
        cnt_ref[0] = 0

    cnt0 = cnt_ref[0]

    @pl.when(cnt0 >= _MAX_OUT)
    def _skip():
        c_ref[0, :, :] = jnp.full((1, _B), cnt0, jnp.int32)

    @pl.when(cnt0 < _MAX_OUT)
    def _work():
        x1r = rowb_ref[0:1, :]
        y1r = rowb_ref[1:2, :]
        x2r = rowb_ref[2:3, :]
        y2r = rowb_ref[3:4, :]
        ar = rowb_ref[4:5, :]

        ext_ref[:, :] = jnp.zeros((1, _B), jnp.float32)
        for q in range(_NB - 1):
            @pl.when(q < p)
            def _pull(q=q):
                s = q * _B
                sf = jnp.where(
                    _iou_tile(coldf_ref[pl.ds(s, _B), 0:1],
                              coldf_ref[pl.ds(s, _B), 1:2],
                              coldf_ref[pl.ds(s, _B), 2:3],
                              coldf_ref[pl.ds(s, _B), 3:4],
                              coldf_ref[pl.ds(s, _B), 4:5],
                              x1r, y1r, x2r, y2r, ar) > thr,
                    1.0, 0.0)
                kq = keep_ref[pl.ds(q, 1), :]
                hits = jnp.dot(kq, sf, preferred_element_type=jnp.float32)
                ext_ref[:, :] = jnp.maximum(ext_ref[:, :], hits)

        lane = lax.broadcasted_iota(jnp.int32, (1, _B), 1)
        valid = jnp.where(p * _B + lane < _N, 1.0, 0.0)
        kb0 = jnp.where(ext_ref[:, :] > 0.0, 0.0, valid)

        x1c = cold_ref[:, 0:1]
        y1c = cold_ref[:, 1:2]
        x2c = cold_ref[:, 2:3]
        y2c = cold_ref[:, 3:4]
        ac = cold_ref[:, 4:5]
        iou_bb = _iou_tile(x1c, y1c, x2c, y2c, ac, x1r, y1r, x2r, y2r, ar)
        tri = (lax.broadcasted_iota(jnp.int32, (_B, _B), 0)
               < lax.broadcasted_iota(jnp.int32, (_B, _B), 1))
        supbb = jnp.where((iou_bb > thr) & tri, 1.0, 0.0)

        def _rcond(carry):
            _, changed = carry
            return changed

        def _rbody(carry):
            kb, _ = carry
            hit = jnp.dot(kb, supbb, preferred_element_type=jnp.float32)
            kb_new = jnp.where(hit > 0.0, 0.0, kb0)
            return kb_new, jnp.any(kb_new != kb)

        kb, _ = lax.while_loop(_rcond, _rbody, (kb0, True))
        keep_ref[pl.ds(p, 1), :] = kb

        tri_le = jnp.where(lax.broadcasted_iota(jnp.int32, (_B, _B), 0)
                           <= lax.broadcasted_iota(jnp.int32, (_B, _B), 1),
                           1.0, 0.0)
        csum = jnp.dot(kb, tri_le, preferred_element_type=jnp.float32)
        c_ref[0, :, :] = csum.astype(jnp.int32) + cnt0
        cnt_ref[0] = cnt0 + jnp.sum(kb).astype(jnp.int32)


def _run_suppression(coldata, rowdata, thr):
    return pl.pallas_call(
        _supp_body,
        grid=(_NB,),
        in_specs=[
            pl.BlockSpec(memory_space=pltpu.SMEM),
            pl.BlockSpec((_NP, 5), lambda p: (0, 0)),
            pl.BlockSpec((_B, 5), lambda p: (p, 0)),
            pl.BlockSpec((5, _B), lambda p: (0, p)),
        ],
        out_specs=pl.BlockSpec((1, 1, _B), lambda p: (p, 0, 0)),
        out_shape=jax.ShapeDtypeStruct((_NB, 1, _B), jnp.int32),
        scratch_shapes=[
            pltpu.VMEM((_NB, _B), jnp.float32),
            pltpu.VMEM((1, _B), jnp.float32),
            pltpu.SMEM((1,), jnp.int32),
        ],
    )(thr, coldata, coldata, rowdata)


def _compact_body(c_hbm, orig_hbm, out_hbm, c_v, o_v, res_v):
    wid = lax.axis_index("s") * _SC_CORES + lax.axis_index("c")
    pltpu.sync_copy(c_hbm, c_v)
    pltpu.sync_copy(orig_hbm, o_v)
    last_idx = jnp.full((16,), _NP - 1, jnp.int32)
    c_last = plsc.load_gather(c_v, [last_idx])
    for g in range(_SC_SLOTS // 16):
        tgt = wid * _SC_SLOTS + g * 16 + lax.iota(jnp.int32, 16) + 1

        def _bstep(_, carry):
            lo, hi = carry
            live = lo < hi
            mid = jnp.minimum((lo + hi) // 2, _NP - 1)
            v = plsc.load_gather(c_v, [mid])
            pred = v >= tgt
            lo2 = jnp.where(live & jnp.logical_not(pred), mid + 1, lo)
            hi2 = jnp.where(live & pred, mid, hi)
            return lo2, hi2

        lo0 = jnp.zeros((16,), jnp.int32)
        hi0 = jnp.full((16,), _NP, jnp.int32)
        pos, _ = lax.fori_loop(0, 13, _bstep, (lo0, hi0))
        valid = c_last >= tgt
        safe = jnp.minimum(pos, _NP - 1)
        ov = plsc.load_gather(o_v, [safe])
        res_v[pl.ds(g * 16, 16)] = jnp.where(valid, ov, -1)
    pltpu.sync_copy(res_v, out_hbm.at[pl.ds(wid * _SC_SLOTS, _SC_SLOTS)])


_CH = _NP // _SC_SUBCORES


def _scatter_body(r1_hbm, r2_hbm, x1_hbm, y1_hbm, x2_hbm, y2_hbm,
                  sx1_hbm, sy1_hbm, sx2_hbm, sy2_hbm, sar_hbm, sorig_hbm,
                  x1v, y1v, x2v, y2v, r1v, r2v, av, ov, riv,
                  shf0, shf1, shf2, shi, sem):
    c = lax.axis_index("c")
    base = lax.axis_index("s") * _CH
    src = pl.ds(base, _CH)
    loads = [pltpu.async_copy(r1_hbm.at[src], r1v, sem),
             pltpu.async_copy(r2_hbm.at[src], r2v, sem),
             pltpu.async_copy(x1_hbm.at[src], x1v, sem),
             pltpu.async_copy(y1_hbm.at[src], y1v, sem),
             pltpu.async_copy(x2_hbm.at[src], x2v, sem),
             pltpu.async_copy(y2_hbm.at[src], y2v, sem)]
    for h in loads:
        h.wait()
    for k in range(_CH // 16):
        sl = pl.ds(k * 16, 16)
        riv[sl] = (r1v[sl] + r2v[sl]).astype(jnp.int32)
        av[sl] = (x2v[sl] - x1v[sl]) * (y2v[sl] - y1v[sl])
        ov[sl] = base + k * 16 + lax.iota(jnp.int32, 16)

    @pl.when(c == 0)
    def _scat0():
        pltpu.sync_copy(x1v, shf0.at[riv])
        pltpu.sync_copy(y1v, shf1.at[riv])
        pltpu.sync_copy(x2v, shf2.at[riv])

    @pl.when(c == 1)
    def _scat1():
        pltpu.sync_copy(y2v, shf0.at[riv])
        pltpu.sync_copy(av, shf1.at[riv])
        pltpu.sync_copy(ov, shi.at[riv])

    plsc.subcore_barrier()

    @pl.when(c == 0)
    def _out0():
        pltpu.sync_copy(shf0.at[src], x1v)
        pltpu.sync_copy(shf1.at[src], y1v)
        pltpu.sync_copy(shf2.at[src], x2v)
        for h in [pltpu.async_copy(x1v, sx1_hbm.at[src], sem),
                  pltpu.async_copy(y1v, sy1_hbm.at[src], sem),
                  pltpu.async_copy(x2v, sx2_hbm.at[src], sem)]:
            h.wait()

    @pl.when(c == 1)
    def _out1():
        pltpu.sync_copy(shf0.at[src], y2v)
        pltpu.sync_copy(shf1.at[src], av)
        pltpu.sync_copy(shi.at[src], ov)
        for h in [pltpu.async_copy(y2v, sy2_hbm.at[src], sem),
                  pltpu.async_copy(av, sar_hbm.at[src], sem),
                  pltpu.async_copy(ov, sorig_hbm.at[src], sem)]:
            h.wait()


@functools.cache
def _scatter_call():
    f32 = jnp.float32
    return pl.kernel(
        _scatter_body,
        out_type=[jax.ShapeDtypeStruct((_NP,), f32)] * 5
        + [jax.ShapeDtypeStruct((_NP,), jnp.int32)],
        mesh=plsc.VectorSubcoreMesh(core_axis_name="c", subcore_axis_name="s"),
        compiler_params=pltpu.CompilerParams(needs_layout_passes=False),
        scratch_types=[
            pltpu.VMEM((_CH,), f32),
            pltpu.VMEM((_CH,), f32),
            pltpu.VMEM((_CH,), f32),
            pltpu.VMEM((_CH,), f32),
            pltpu.VMEM((_CH,), f32),
            pltpu.VMEM((_CH,), f32),
            pltpu.VMEM((_CH,), f32),
            pltpu.VMEM((_CH,), jnp.int32),
            pltpu.VMEM((_CH,), jnp.int32),
            pltpu.VMEM_SHARED((_NP,), f32),
            pltpu.VMEM_SHARED((_NP,), f32),
            pltpu.VMEM_SHARED((_NP,), f32),
            pltpu.VMEM_SHARED((_NP,), jnp.int32),
            pltpu.SemaphoreType.DMA,
        ],
    )


@functools.cache
def _compact_call():
    return pl.kernel(
        _compact_body,
        out_type=jax.ShapeDtypeStruct((_OUT_PAD,), jnp.int32),
        mesh=plsc.VectorSubcoreMesh(core_axis_name="c", subcore_axis_name="s"),
        compiler_params=pltpu.CompilerParams(needs_layout_passes=False),
        scratch_types=[
            pltpu.VMEM((_NP,), jnp.int32),
            pltpu.VMEM((_NP,), jnp.int32),
            pltpu.VMEM((_SC_SLOTS,), jnp.int32),
        ],
    )


def kernel(boxes, scores, iou_threshold):
    pad = _NP - _N
    sp = jnp.pad(scores.astype(jnp.float32), (0, pad), constant_values=-1.0)
    a1, a2 = _run_rank(sp.reshape(1, _NP), sp.reshape(_NP, 1))
    x1u = jnp.pad(boxes[:, 0], (0, pad))
    y1u = jnp.pad(boxes[:, 1], (0, pad))
    x2u = jnp.pad(boxes[:, 2], (0, pad))
    y2u = jnp.pad(boxes[:, 3], (0, pad))
    sx1, sy1, sx2, sy2, sar, sorig = _scatter_call()(
        a1.reshape(_NP), a2.reshape(_NP), x1u, y1u, x2u, y2u)
    rowdata = jnp.stack([sx1, sy1, sx2, sy2, sar])
    coldata = rowdata.T
    thr = jnp.reshape(iou_threshold.astype(jnp.float32), (1,))

    c = _run_suppression(coldata, rowdata, thr).reshape(_NP)
    out = _compact_call()(c, sorig)
    return out[:_MAX_OUT]
